# baseline (speedup 1.0000x reference)
import jax
import jax.numpy as jnp
from jax import lax
from jax.experimental import pallas as pl
from jax.experimental.pallas import tpu as pltpu
from jax.experimental.pallas import tpu_sc as plsc
import functools

VOCAB = 1000000
EMB = 64
B = 16384
L = 50

NC = 2
NS = 16
NW = NC * NS

N = B * L
G = 128
PER_W = N // NW
NG = PER_W // G
NBUF = 8


def _make_gather():
    mesh = plsc.VectorSubcoreMesh(
        core_axis_name="c", subcore_axis_name="s",
        num_cores=NC, num_subcores=NS)

    @functools.partial(
        pl.kernel,
        out_type=jax.ShapeDtypeStruct((NW, NG, G // 2, 2 * EMB), jnp.float32),
        mesh=mesh,
        scratch_types=[
            pltpu.VMEM((NG, G), jnp.int32),
            pltpu.VMEM((NBUF, G, EMB), jnp.float32),
            pltpu.SemaphoreType.DMA((NBUF,)),
            pltpu.SemaphoreType.DMA((NBUF,)),
        ],
        compiler_params=pltpu.CompilerParams(use_tc_tiling_on_sc=False),
    )
    def gather_kernel(idx_hbm, table_hbm, out_hbm, idx_v, rows_v, gsem, osem):
        cid = lax.axis_index("c")
        sid = lax.axis_index("s")
        wid = sid * NC + cid
        pltpu.sync_copy(idx_hbm.at[wid], idx_v)

        def start_gather(b, j):
            pltpu.make_async_copy(
                table_hbm.at[idx_v.at[j]], rows_v.at[b], gsem.at[b]).start()

        def wait_gather(b, j):
            pltpu.make_async_copy(
                table_hbm.at[idx_v.at[j]], rows_v.at[b], gsem.at[b]).wait()

        def start_out(b, j):
            pltpu.make_async_copy(
                rows_v.at[b, pl.ds(0, G // 2)],
                out_hbm.at[wid, j, :, pl.ds(0, EMB)], osem.at[b]).start()
            pltpu.make_async_copy(
                rows_v.at[b, pl.ds(G // 2, G // 2)],
                out_hbm.at[wid, j, :, pl.ds(EMB, EMB)], osem.at[b]).start()

        def wait_out(b, j):
            pltpu.make_async_copy(
                rows_v.at[b, pl.ds(0, G // 2)],
                out_hbm.at[wid, j, :, pl.ds(0, EMB)], osem.at[b]).wait()
            pltpu.make_async_copy(
                rows_v.at[b, pl.ds(G // 2, G // 2)],
                out_hbm.at[wid, j, :, pl.ds(EMB, EMB)], osem.at[b]).wait()

        for b in range(NBUF):
            start_gather(b, b)

        @pl.loop(0, NG - NBUF, step=NBUF)
        def _(j0):
            for b in range(NBUF):
                wait_gather(b, j0 + b)
                start_out(b, j0 + b)
            for b in range(NBUF):
                wait_out(b, j0 + b)
                start_gather(b, j0 + NBUF + b)

        for b in range(NBUF):
            wait_gather(b, NG - NBUF + b)
            start_out(b, NG - NBUF + b)
        for b in range(NBUF):
            wait_out(b, NG - NBUF + b)

    return gather_kernel


_gather = _make_gather()


def kernel(text_vec, w2v_table):
    idx = (text_vec.astype(jnp.int32)
           .reshape(NW, NG, G // 2, 2)
           .transpose(0, 1, 3, 2)
           .reshape(NW, NG, G))
    out = _gather(idx, w2v_table)
    return out.reshape(B, L, EMB)
